# initial kernel scaffold (unmeasured)
import jax
import jax.numpy as jnp
from jax import lax
from jax.experimental import pallas as pl
from jax.experimental.pallas import tpu as pltpu

M = 4096
N = 4096
BLK = M // 8


def _allreduce_body(own_ref, send_ref, out_ref, rs_buf, send_sems, recv_sems):
    x = lax.axis_index("x")
    y = lax.axis_index("y")
    z = lax.axis_index("z")
    m = 2 * y + z
    b_own = 4 * x + m

    barrier = pltpu.get_barrier_semaphore()
    for nbr in ((1 - x, y, z), (x, 1 - y, z), (x, y, 1 - z)):
        pl.semaphore_signal(barrier, inc=1, device_id=nbr,
                            device_id_type=pl.DeviceIdType.MESH)
    pl.semaphore_wait(barrier, 3)

    rs = pltpu.make_async_remote_copy(
        src_ref=send_ref,
        dst_ref=rs_buf,
        send_sem=send_sems.at[0],
        recv_sem=recv_sems.at[0],
        device_id=(1 - x, y, z),
        device_id_type=pl.DeviceIdType.MESH,
    )
    rs.start()
    rs.wait()
    out_ref[pl.ds(b_own * BLK, BLK), :] = own_ref[...] + rs_buf[...]

    hops = (
        (1, b_own, (x, y, 1 - z)),
        (2, 4 * x + 2 * y, (x, 1 - y, z)),
        (4, 4 * x, (1 - x, y, z)),
    )
    for i, (nblk, start_blk, nbr) in enumerate(hops):
        rows = pl.ds(start_blk * BLK, nblk * BLK)
        ag = pltpu.make_async_remote_copy(
            src_ref=out_ref.at[rows, :],
            dst_ref=out_ref.at[rows, :],
            send_sem=send_sems.at[i + 1],
            recv_sem=recv_sems.at[i + 1],
            device_id=nbr,
            device_id_type=pl.DeviceIdType.MESH,
        )
        ag.start()
        ag.wait()


def _allreduce(own, send):
    return pl.pallas_call(
        _allreduce_body,
        out_shape=jax.ShapeDtypeStruct((M, N), jnp.float32),
        in_specs=[
            pl.BlockSpec(memory_space=pltpu.VMEM),
            pl.BlockSpec(memory_space=pltpu.VMEM),
        ],
        out_specs=pl.BlockSpec(memory_space=pltpu.VMEM),
        scratch_shapes=[
            pltpu.VMEM((BLK, N), jnp.float32),
            pltpu.SemaphoreType.DMA((4,)),
            pltpu.SemaphoreType.DMA((4,)),
        ],
        compiler_params=pltpu.CompilerParams(collective_id=0),
    )(own, send)


def kernel(dy, W):
    x = lax.axis_index("x")
    y = lax.axis_index("y")
    z = lax.axis_index("z")
    m = 2 * y + z

    lo = lax.dynamic_slice_in_dim(dy, m * BLK, BLK, axis=0)
    hi = lax.dynamic_slice_in_dim(dy, (m + 4) * BLK, BLK, axis=0)
    p_lo = lo @ W.T
    p_hi = hi @ W.T

    own = jnp.where(x == 0, p_lo, p_hi)
    send = jnp.where(x == 0, p_hi, p_lo)
    return _allreduce(own, send)


# baseline (device time: 914765 ns/iter reference)
import jax
import jax.numpy as jnp
from jax import lax
from jax.experimental import pallas as pl
from jax.experimental.pallas import tpu as pltpu

M = 4096
N = 4096
BLK = M // 8


def _allreduce_body(own_ref, send_ref, out_ref, rs_buf, send_sems, recv_sems,
                    copy_sem):
    x = lax.axis_index("x")
    y = lax.axis_index("y")
    z = lax.axis_index("z")
    m = 2 * y + z
    b_own = 4 * x + m

    barrier = pltpu.get_barrier_semaphore()
    for nbr in ((1 - x, y, z), (x, 1 - y, z), (x, y, 1 - z)):
        pl.semaphore_signal(barrier, inc=1, device_id=nbr,
                            device_id_type=pl.DeviceIdType.MESH)
    pl.semaphore_wait(barrier, 3)

    rs = pltpu.make_async_remote_copy(
        src_ref=send_ref,
        dst_ref=rs_buf,
        send_sem=send_sems.at[0],
        recv_sem=recv_sems.at[0],
        device_id=(1 - x, y, z),
        device_id_type=pl.DeviceIdType.MESH,
    )
    rs.start()
    rs.wait()
    rs_buf[...] = own_ref[...] + rs_buf[...]
    cp = pltpu.make_async_copy(
        rs_buf, out_ref.at[pl.ds(b_own * BLK, BLK), :], copy_sem)
    cp.start()
    cp.wait()

    hops = (
        (1, b_own, (x, y, 1 - z)),
        (2, 4 * x + 2 * y, (x, 1 - y, z)),
        (4, 4 * x, (1 - x, y, z)),
    )
    for i, (nblk, start_blk, nbr) in enumerate(hops):
        rows = pl.ds(start_blk * BLK, nblk * BLK)
        ag = pltpu.make_async_remote_copy(
            src_ref=out_ref.at[rows, :],
            dst_ref=out_ref.at[rows, :],
            send_sem=send_sems.at[i + 1],
            recv_sem=recv_sems.at[i + 1],
            device_id=nbr,
            device_id_type=pl.DeviceIdType.MESH,
        )
        ag.start()
        ag.wait()


def _allreduce(own, send):
    return pl.pallas_call(
        _allreduce_body,
        out_shape=jax.ShapeDtypeStruct((M, N), jnp.float32),
        in_specs=[
            pl.BlockSpec(memory_space=pltpu.VMEM),
            pl.BlockSpec(memory_space=pltpu.VMEM),
        ],
        out_specs=pl.BlockSpec(memory_space=pl.ANY),
        scratch_shapes=[
            pltpu.VMEM((BLK, N), jnp.float32),
            pltpu.SemaphoreType.DMA((4,)),
            pltpu.SemaphoreType.DMA((4,)),
            pltpu.SemaphoreType.DMA,
        ],
        compiler_params=pltpu.CompilerParams(collective_id=0),
    )(own, send)


def kernel(dy, W):
    x = lax.axis_index("x")
    y = lax.axis_index("y")
    z = lax.axis_index("z")
    m = 2 * y + z

    lo = lax.dynamic_slice_in_dim(dy, m * BLK, BLK, axis=0)
    hi = lax.dynamic_slice_in_dim(dy, (m + 4) * BLK, BLK, axis=0)
    p_lo = lo @ W.T
    p_hi = hi @ W.T

    own = jnp.where(x == 0, p_lo, p_hi)
    send = jnp.where(x == 0, p_hi, p_lo)
    return _allreduce(own, send)


# device time: 503332 ns/iter; 1.8174x vs baseline; 1.8174x over previous
import jax
import jax.numpy as jnp
from jax import lax
from jax.experimental import pallas as pl
from jax.experimental.pallas import tpu as pltpu

M = 4096
N = 4096
BLK = M // 8

AXIS_W = {"x": 4, "y": 2, "z": 1}
SCHEDULES = (("z", "y", "x"), ("y", "x", "z"), ("x", "z", "y"))
COL0 = (0, 1408, 2816)
COLW = (1408, 1408, 1280)
N_SEMS = 16


def _runs(exchanged):
    offs = [0]
    for a in exchanged:
        w = AXIS_W[a]
        offs = sorted(o + d for o in offs for d in (0, w))
    runs = []
    start, length = offs[0], 1
    for o in offs[1:]:
        if o == start + length:
            length += 1
        else:
            runs.append((start, length))
            start, length = o, 1
    runs.append((start, length))
    return runs


def _allreduce_body(own_ref, send_ref, out_ref, rs_buf, send_sems, recv_sems,
                    copy_sem):
    x = lax.axis_index("x")
    y = lax.axis_index("y")
    z = lax.axis_index("z")
    idx = {"x": x, "y": y, "z": z}
    m = 2 * y + z
    b_own = 4 * x + m

    def nbr_of(a):
        return tuple(1 - idx[ax] if ax == a else idx[ax] for ax in "xyz")

    barrier = pltpu.get_barrier_semaphore()
    for a in "xyz":
        pl.semaphore_signal(barrier, inc=1, device_id=nbr_of(a),
                            device_id_type=pl.DeviceIdType.MESH)
    pl.semaphore_wait(barrier, 3)

    rs = pltpu.make_async_remote_copy(
        src_ref=send_ref,
        dst_ref=rs_buf,
        send_sem=send_sems.at[0],
        recv_sem=recv_sems.at[0],
        device_id=nbr_of("x"),
        device_id_type=pl.DeviceIdType.MESH,
    )
    rs.start()
    rs.wait()
    rs_buf[...] = own_ref[...] + rs_buf[...]
    cp = pltpu.make_async_copy(
        rs_buf, out_ref.at[pl.ds(b_own * BLK, BLK), :], copy_sem)
    cp.start()
    cp.wait()

    sem_i = 1
    exchanged = [(), (), ()]
    for h in range(3):
        descs = []
        for s, order in enumerate(SCHEDULES):
            a = order[h]
            ex = exchanged[s]
            base = sum(AXIS_W[ax] * idx[ax] for ax in "xyz" if ax not in ex)
            cols = pl.ds(COL0[s], COLW[s])
            for off, length in _runs(ex):
                rows = pl.ds((base + off) * BLK, length * BLK)
                d = pltpu.make_async_remote_copy(
                    src_ref=out_ref.at[rows, cols],
                    dst_ref=out_ref.at[rows, cols],
                    send_sem=send_sems.at[sem_i],
                    recv_sem=recv_sems.at[sem_i],
                    device_id=nbr_of(a),
                    device_id_type=pl.DeviceIdType.MESH,
                )
                d.start()
                descs.append(d)
                sem_i += 1
            exchanged[s] = ex + (a,)
        for d in descs:
            d.wait()


def _allreduce(own, send):
    return pl.pallas_call(
        _allreduce_body,
        out_shape=jax.ShapeDtypeStruct((M, N), jnp.float32),
        in_specs=[
            pl.BlockSpec(memory_space=pltpu.VMEM),
            pl.BlockSpec(memory_space=pltpu.VMEM),
        ],
        out_specs=pl.BlockSpec(memory_space=pl.ANY),
        scratch_shapes=[
            pltpu.VMEM((BLK, N), jnp.float32),
            pltpu.SemaphoreType.DMA((N_SEMS,)),
            pltpu.SemaphoreType.DMA((N_SEMS,)),
            pltpu.SemaphoreType.DMA,
        ],
        compiler_params=pltpu.CompilerParams(collective_id=0),
    )(own, send)


def kernel(dy, W):
    x = lax.axis_index("x")
    y = lax.axis_index("y")
    z = lax.axis_index("z")
    m = 2 * y + z

    lo = lax.dynamic_slice_in_dim(dy, m * BLK, BLK, axis=0)
    hi = lax.dynamic_slice_in_dim(dy, (m + 4) * BLK, BLK, axis=0)
    p_lo = lo @ W.T
    p_hi = hi @ W.T

    own = jnp.where(x == 0, p_lo, p_hi)
    send = jnp.where(x == 0, p_hi, p_lo)
    return _allreduce(own, send)
